# baseline (speedup 1.0000x reference)
import numpy as np

import jax
import jax.numpy as jnp
from jax.experimental import pallas as pl
from jax.experimental.pallas import tpu as pltpu

B, C_IN, H, W = 64, 512, 28, 28
HW = H * W
C_MID, GROWTH = 128, 32
C_OUT = C_IN + GROWTH
EPS = 1e-5
QMAX = 255.0
NUM_CHUNKS = 16
GB = 4
NG = B // GB
K2 = 9 * C_MID
K2A = K2 + 16


def _qparams(x):
    mn = x.min()
    mx = x.max()
    scale = jnp.maximum((mx - mn) / QMAX, 1e-8)
    return mn, scale


def _stats_kernel(x_ref, o_ref):
    x0, x1, x2, x3 = x_ref[0], x_ref[1], x_ref[2], x_ref[3]
    s = (x0 + x1) + (x2 + x3)
    ss = (x0 * x0 + x1 * x1) + (x2 * x2 + x3 * x3)
    mn = jnp.minimum(jnp.minimum(x0, x1), jnp.minimum(x2, x3))
    mx = jnp.maximum(jnp.maximum(x0, x1), jnp.maximum(x2, x3))
    o_ref[0, 0] = jnp.sum(s, axis=1)
    o_ref[0, 1] = jnp.sum(ss, axis=1)
    o_ref[0, 2] = jnp.min(mn, axis=1)
    o_ref[0, 3] = jnp.max(mx, axis=1)


def _conv1_kernel(x_ref, ap_ref, tp_ref, w1_ref, rc_ref, q_ref,
                  big_ref, h2_ref, st_ref):
    m1p, c1, c2 = q_ref[0], q_ref[1], q_ref[2]
    ap = ap_ref[...]
    tp = tp_ref[...]
    rc = rc_ref[...]
    s_a = mn_a = mx_a = None
    for k in range(GB):
        xb = x_ref[k]
        v = jnp.maximum(xb * ap + tp, 0.0) - m1p
        q1 = jnp.clip(jnp.round(v), 0.0, QMAX).astype(jnp.bfloat16)
        s_all = jnp.dot(w1_ref[...], q1,
                        preferred_element_type=jnp.float32)
        h2 = c1 * s_all[:C_MID] + c2 * s_all[C_MID:C_MID + 1] + rc
        h2b = h2.astype(jnp.bfloat16)
        h2_ref[k] = h2b
        big_ref[k, :C_IN] = xb
        h2f = h2b.astype(jnp.float32)
        s_k = jnp.sum(h2f, axis=1)
        mn_k = jnp.min(h2f, axis=1)
        mx_k = jnp.max(h2f, axis=1)
        if k == 0:
            s_a, mn_a, mx_a = s_k, mn_k, mx_k
        else:
            s_a = s_a + s_k
            mn_a = jnp.minimum(mn_a, mn_k)
            mx_a = jnp.maximum(mx_a, mx_k)
    st_ref[0, 0] = s_a
    st_ref[0, 1] = mn_a
    st_ref[0, 2] = mx_a
    st_ref[0, 3] = jnp.zeros_like(s_a)


_OFFS = [(ky - 1, kx - 1) for ky in range(3) for kx in range(3)]
_PAD = 32


def _conv2_kernel(big_in_ref, h2_ref, a_ref, c_ref, w2_ref, q_ref,
                  o_ref, f0_scr, fl_scr, fr_scr, g_scr):
    del big_in_ref
    m3p = q_ref[0]
    a3 = a_ref[...]
    c3 = c_ref[...]

    @pl.when(pl.program_id(0) == 0)
    def _init():
        zpad = jnp.zeros((C_MID, _PAD), jnp.bfloat16)
        for f in (f0_scr, fl_scr, fr_scr):
            f[:, :_PAD] = zpad
            f[:, _PAD + HW:] = jnp.zeros((C_MID, 896 - _PAD - HW),
                                         jnp.bfloat16)
        p = jax.lax.broadcasted_iota(jnp.int32, (1, HW), 1)
        y, xc = p // W, p % W
        for i, (dy, dx) in enumerate(_OFFS):
            ok = (y + dy >= 0) & (y + dy < H) & (xc + dx >= 0) & (xc + dx < W)
            g_scr[K2 + i:K2 + i + 1, :] = jnp.where(
                ok, jnp.float32(1), jnp.float32(0)).astype(jnp.bfloat16)
        g_scr[K2 + 9:, :] = jnp.zeros((K2A - K2 - 9, HW), jnp.bfloat16)

    xc = jax.lax.broadcasted_iota(jnp.int32, (1, HW), 1) % W
    for k in range(GB):
        h2 = h2_ref[k].astype(jnp.float32)
        v = jnp.maximum(h2 * a3 + c3, 0.0) - m3p
        q3 = jnp.clip(jnp.round(v), 0.0, QMAX).astype(jnp.bfloat16)
        f0_scr[:, _PAD:_PAD + HW] = q3
        fl_scr[:, _PAD:_PAD + HW] = jnp.where(xc == 0, jnp.bfloat16(0), q3)
        fr_scr[:, _PAD:_PAD + HW] = jnp.where(xc == W - 1, jnp.bfloat16(0), q3)
        for i, (dy, dx) in enumerate(_OFFS):
            f = f0_scr if dx == 0 else (fl_scr if dx == 1 else fr_scr)
            off = _PAD + dy * W + dx
            g_scr[i * C_MID:(i + 1) * C_MID, :] = f[:, off:off + HW]
        o_ref[k, 0] = jnp.dot(w2_ref[...], g_scr[...],
                              preferred_element_type=jnp.float32)


def kernel(x, bn1_w, bn1_b, conv1_w, rbn_w, rbn_b, conv2_w):
    xr = x.reshape(B, C_IN, HW)
    f32 = jnp.float32

    st1 = pl.pallas_call(
        _stats_kernel,
        grid=(NG,),
        in_specs=[pl.BlockSpec((GB, C_IN, HW), lambda i: (i, 0, 0))],
        out_specs=pl.BlockSpec((1, 4, C_IN), lambda i: (i, 0, 0)),
        out_shape=jax.ShapeDtypeStruct((NG, 4, C_IN), f32),
        compiler_params=pltpu.CompilerParams(
            dimension_semantics=("arbitrary",)),
        name="dense_stats",
    )(xr)

    n1 = float(B * HW)
    mean1 = jnp.sum(st1[:, 0], axis=0) / n1
    var1 = jnp.sum(st1[:, 1], axis=0) / n1 - mean1 * mean1
    a1 = bn1_w * jax.lax.rsqrt(var1 + EPS)
    t1 = bn1_b - mean1 * a1
    cmin = jnp.min(st1[:, 2], axis=0)
    cmax = jnp.max(st1[:, 3], axis=0)
    lo = a1 * cmin + t1
    hi = a1 * cmax + t1
    mn1 = jnp.maximum(jnp.min(lo), 0.0)
    mx1 = jnp.maximum(jnp.max(hi), 0.0)
    sc1 = jnp.maximum((mx1 - mn1) / QMAX, 1e-8)
    inv1 = 1.0 / sc1

    w1 = conv1_w.reshape(C_MID, C_IN)
    mnw1, sw1 = _qparams(w1)
    qw1 = jnp.round((w1 - mnw1) / sw1)
    w1aug = jnp.concatenate(
        [qw1, jnp.ones((1, C_IN), f32), jnp.zeros((7, C_IN), f32)],
        axis=0).astype(jnp.bfloat16)
    rowsum_qw1 = jnp.sum(qw1, axis=1)
    rc = (sw1 * mn1) * rowsum_qw1 + (C_IN * mnw1) * mn1
    qv1 = jnp.stack([mn1 * inv1, sw1 * sc1, mnw1 * sc1])

    big_shape = jax.ShapeDtypeStruct((B, C_OUT, HW), f32)
    vspec_in = pl.BlockSpec((C_IN, 1), lambda i: (0, 0))
    vspec_mid = pl.BlockSpec((C_MID, 1), lambda i: (0, 0))
    big, h2, st2 = pl.pallas_call(
        _conv1_kernel,
        grid=(NG,),
        in_specs=[
            pl.BlockSpec((GB, C_IN, HW), lambda i: (i, 0, 0)),
            vspec_in, vspec_in,
            pl.BlockSpec((C_MID + 8, C_IN), lambda i: (0, 0)),
            vspec_mid,
            pl.BlockSpec(memory_space=pltpu.SMEM),
        ],
        out_specs=[
            pl.BlockSpec((GB, C_OUT, HW), lambda i: (i, 0, 0)),
            pl.BlockSpec((GB, C_MID, HW), lambda i: (i, 0, 0)),
            pl.BlockSpec((1, 4, C_MID), lambda i: (i, 0, 0)),
        ],
        out_shape=[
            big_shape,
            jax.ShapeDtypeStruct((B, C_MID, HW), jnp.bfloat16),
            jax.ShapeDtypeStruct((NG, 4, C_MID), f32),
        ],
        compiler_params=pltpu.CompilerParams(
            dimension_semantics=("arbitrary",),
            vmem_limit_bytes=56 * 1024 * 1024),
        name="dense_conv1",
    )(xr, (a1 * inv1)[:, None], (t1 * inv1)[:, None], w1aug,
      rc[:, None], qv1)

    mean2 = jnp.sum(st2[:, 0], axis=0) / n1
    mean_min = jnp.mean(st2[:, 1], axis=0)
    mean_max = jnp.mean(st2[:, 2], axis=0)
    n_chunk = float(B * HW // NUM_CHUNKS)
    scale_fix = ((0.5 * 0.35) * (1.0 + (np.pi * np.log(4.0)) ** 0.5)
                 / ((2.0 * np.log(n_chunk)) ** 0.5))
    scale2 = 1.0 / ((mean_max - mean_min) * scale_fix + EPS)

    def _qdq(t):
        mn, sc = _qparams(t)
        return jnp.round((t - mn) / sc) * sc + mn

    a2 = scale2 * _qdq(rbn_w)
    c2 = _qdq(rbn_b) - mean2 * a2

    gmin2 = jnp.min(st2[:, 1], axis=0)
    gmax2 = jnp.max(st2[:, 2], axis=0)
    lo3 = a2 * gmin2 + c2
    hi3 = a2 * gmax2 + c2
    mn3 = jnp.maximum(jnp.min(lo3), 0.0)
    mx3 = jnp.maximum(jnp.max(hi3), 0.0)
    sc3 = jnp.maximum((mx3 - mn3) / QMAX, 1e-8)
    inv3 = 1.0 / sc3

    w2q = _qdq(conv2_w)
    w2mat = w2q.transpose(0, 2, 3, 1).reshape(GROWTH, K2)
    ws_eff = jnp.sum(w2q, axis=1).reshape(GROWTH, 9)
    w2aug = jnp.concatenate(
        [sc3 * w2mat, mn3 * ws_eff, jnp.zeros((GROWTH, K2A - K2 - 9), f32)],
        axis=1).astype(jnp.bfloat16)
    qv3 = jnp.stack([mn3 * inv3])

    out = pl.pallas_call(
        _conv2_kernel,
        grid=(NG,),
        in_specs=[
            pl.BlockSpec(memory_space=pl.ANY),
            pl.BlockSpec((GB, C_MID, HW), lambda i: (i, 0, 0)),
            vspec_mid, vspec_mid,
            pl.BlockSpec((GROWTH, K2A), lambda i: (0, 0)),
            pl.BlockSpec(memory_space=pltpu.SMEM),
        ],
        out_specs=pl.BlockSpec((GB, 1, GROWTH, HW), lambda i: (i, 16, 0, 0)),
        out_shape=jax.ShapeDtypeStruct((B, 17, GROWTH, HW), f32),
        scratch_shapes=[
            pltpu.VMEM((C_MID, 896), jnp.bfloat16),
            pltpu.VMEM((C_MID, 896), jnp.bfloat16),
            pltpu.VMEM((C_MID, 896), jnp.bfloat16),
            pltpu.VMEM((K2A, HW), jnp.bfloat16),
        ],
        input_output_aliases={0: 0},
        compiler_params=pltpu.CompilerParams(
            dimension_semantics=("arbitrary",),
            vmem_limit_bytes=56 * 1024 * 1024),
        name="dense_conv2",
    )(big.reshape(B, 17, GROWTH, HW), h2,
      (a2 * inv3)[:, None], (c2 * inv3)[:, None], w2aug, qv3)

    return out.reshape(B, C_OUT, H, W)


# speedup vs baseline: 3.3411x; 3.3411x over previous
import numpy as np

import jax
import jax.numpy as jnp
from jax.experimental import pallas as pl
from jax.experimental.pallas import tpu as pltpu

B, C_IN, H, W = 64, 512, 28, 28
HW = H * W
R_ALL = H * W * B
C_MID, GROWTH = 128, 32
C_OUT = C_IN + GROWTH
EPS = 1e-5
QMAX = 255.0
NUM_CHUNKS = 16
RB = 3584
NB = R_ALL // RB
HL = W * B
MARG = HL + B
SROWS = MARG + RB + MARG
K2 = 9 * C_MID


def _qparams(x):
    mn = x.min()
    mx = x.max()
    scale = jnp.maximum((mx - mn) / QMAX, 1e-8)
    return mn, scale


def _stats_kernel(x_ref, o_ref):
    xb = x_ref[...]
    o_ref[0, 0] = jnp.sum(xb, axis=0)
    o_ref[0, 1] = jnp.sum(xb * xb, axis=0)
    o_ref[0, 2] = jnp.min(xb, axis=0)
    o_ref[0, 3] = jnp.max(xb, axis=0)


def _conv1_kernel(x_ref, ap_ref, tp_ref, w1_ref, rc_ref, q_ref,
                  h2_ref, st_ref):
    m1p, c1, c2 = q_ref[0], q_ref[1], q_ref[2]
    xb = x_ref[...]
    v = jnp.maximum(xb * ap_ref[...] + tp_ref[...], 0.0) - m1p
    q1 = jnp.clip(jnp.round(v), 0.0, QMAX).astype(jnp.bfloat16)
    s = jnp.dot(q1, w1_ref[...],
                preferred_element_type=jnp.float32)
    h2 = c1 * s[:, :C_MID] + c2 * s[:, C_MID:C_MID + 1] + rc_ref[...]
    h2b = h2.astype(jnp.bfloat16)
    h2_ref[...] = h2b
    h2r = h2b.astype(jnp.float32).reshape(RB // B, B, C_MID)
    st_ref[0, 0] = jnp.sum(h2r, axis=0)
    st_ref[0, 1] = jnp.min(h2r, axis=0)
    st_ref[0, 2] = jnp.max(h2r, axis=0)
    st_ref[0, 3] = jnp.zeros((B, C_MID), jnp.float32)


def _conv2_kernel(x_ref, hp_ref, hc_ref, hn_ref, a_ref, c_ref, w2_ref,
                  q_ref, o_ref, s_scr, g_scr):
    i = pl.program_id(0)
    m3p, sc3, mn3 = q_ref[0], q_ref[1], q_ref[2]

    s_scr[:MARG] = hp_ref[RB - MARG:]
    s_scr[MARG:MARG + RB] = hc_ref[...]
    s_scr[MARG + RB:] = hn_ref[:MARG]
    sf = s_scr[...].astype(jnp.float32)
    v = jnp.maximum(sf * a_ref[...] + c_ref[...], 0.0) - m3p
    q3 = jnp.clip(jnp.round(v), 0.0, QMAX)
    s_scr[...] = (q3 * sc3 + mn3).astype(jnp.bfloat16)

    for o_i in range(9):
        dy, dx = o_i // 3 - 1, o_i % 3 - 1
        ofs = MARG + dy * HL + dx * B
        g_scr[:, o_i * C_MID:(o_i + 1) * C_MID] = s_scr[ofs:ofs + RB]

    zb = jnp.zeros((B, C_MID), jnp.bfloat16)
    for o_i in (0, 3, 6):
        oc = slice(o_i * C_MID, (o_i + 1) * C_MID)
        g_scr[0:B, oc] = zb
        g_scr[HL:HL + B, oc] = zb
    for o_i in (2, 5, 8):
        oc = slice(o_i * C_MID, (o_i + 1) * C_MID)
        g_scr[HL - B:HL, oc] = zb
        g_scr[RB - B:RB, oc] = zb

    @pl.when(i == 0)
    def _first():
        g_scr[0:HL, 0:3 * C_MID] = jnp.zeros((HL, 3 * C_MID), jnp.bfloat16)

    @pl.when(i == NB - 1)
    def _last():
        g_scr[HL:RB, 6 * C_MID:] = jnp.zeros((HL, 3 * C_MID), jnp.bfloat16)

    h4 = jnp.dot(g_scr[...], w2_ref[...],
                 preferred_element_type=jnp.float32)
    o_ref[:, :C_IN] = x_ref[...]
    o_ref[:, C_IN:] = h4


def kernel(x, bn1_w, bn1_b, conv1_w, rbn_w, rbn_b, conv2_w):
    xt = x.transpose(2, 3, 0, 1).reshape(R_ALL, C_IN)
    f32 = jnp.float32

    st1 = pl.pallas_call(
        _stats_kernel,
        grid=(NB,),
        in_specs=[pl.BlockSpec((RB, C_IN), lambda i: (i, 0))],
        out_specs=pl.BlockSpec((1, 4, C_IN), lambda i: (i, 0, 0)),
        out_shape=jax.ShapeDtypeStruct((NB, 4, C_IN), f32),
        compiler_params=pltpu.CompilerParams(
            dimension_semantics=("arbitrary",),
            vmem_limit_bytes=56 * 1024 * 1024),
        name="dense_stats",
    )(xt)

    n1 = float(R_ALL)
    mean1 = jnp.sum(st1[:, 0], axis=0) / n1
    var1 = jnp.sum(st1[:, 1], axis=0) / n1 - mean1 * mean1
    a1 = bn1_w * jax.lax.rsqrt(var1 + EPS)
    t1 = bn1_b - mean1 * a1
    lo = a1 * jnp.min(st1[:, 2], axis=0) + t1
    hi = a1 * jnp.max(st1[:, 3], axis=0) + t1
    mn1 = jnp.maximum(jnp.min(lo), 0.0)
    mx1 = jnp.maximum(jnp.max(hi), 0.0)
    sc1 = jnp.maximum((mx1 - mn1) / QMAX, 1e-8)
    inv1 = 1.0 / sc1

    w1 = conv1_w.reshape(C_MID, C_IN)
    mnw1, sw1 = _qparams(w1)
    qw1 = jnp.round((w1 - mnw1) / sw1)
    w1aug = jnp.concatenate(
        [qw1.T, jnp.ones((C_IN, 1), f32), jnp.zeros((C_IN, 7), f32)],
        axis=1).astype(jnp.bfloat16)
    rc = (sw1 * mn1) * jnp.sum(qw1, axis=1) + (C_IN * mnw1) * mn1
    qv1 = jnp.stack([mn1 * inv1, sw1 * sc1, mnw1 * sc1])

    rvec = lambda v: v.reshape(1, -1)
    h2, st2 = pl.pallas_call(
        _conv1_kernel,
        grid=(NB,),
        in_specs=[
            pl.BlockSpec((RB, C_IN), lambda i: (i, 0)),
            pl.BlockSpec((1, C_IN), lambda i: (0, 0)),
            pl.BlockSpec((1, C_IN), lambda i: (0, 0)),
            pl.BlockSpec((C_IN, C_MID + 8), lambda i: (0, 0)),
            pl.BlockSpec((1, C_MID), lambda i: (0, 0)),
            pl.BlockSpec(memory_space=pltpu.SMEM),
        ],
        out_specs=[
            pl.BlockSpec((RB, C_MID), lambda i: (i, 0)),
            pl.BlockSpec((1, 4, B, C_MID), lambda i: (i, 0, 0, 0)),
        ],
        out_shape=[
            jax.ShapeDtypeStruct((R_ALL, C_MID), jnp.bfloat16),
            jax.ShapeDtypeStruct((NB, 4, B, C_MID), f32),
        ],
        compiler_params=pltpu.CompilerParams(
            dimension_semantics=("arbitrary",),
            vmem_limit_bytes=56 * 1024 * 1024),
        name="dense_conv1",
    )(xt, rvec(a1 * inv1), rvec(t1 * inv1), w1aug, rvec(rc), qv1)

    bsum = jnp.sum(st2[:, 0], axis=0)
    bmin = jnp.min(st2[:, 1], axis=0)
    bmax = jnp.max(st2[:, 2], axis=0)
    mean2 = jnp.sum(bsum, axis=0) / n1
    mean_min = jnp.mean(jnp.min(bmin.reshape(NUM_CHUNKS, 4, C_MID), 1), 0)
    mean_max = jnp.mean(jnp.max(bmax.reshape(NUM_CHUNKS, 4, C_MID), 1), 0)
    n_chunk = float(R_ALL // NUM_CHUNKS)
    scale_fix = ((0.5 * 0.35) * (1.0 + (np.pi * np.log(4.0)) ** 0.5)
                 / ((2.0 * np.log(n_chunk)) ** 0.5))
    scale2 = 1.0 / ((mean_max - mean_min) * scale_fix + EPS)

    def _qdq(t):
        mn, sc = _qparams(t)
        return jnp.round((t - mn) / sc) * sc + mn

    a2 = scale2 * _qdq(rbn_w)
    c2 = _qdq(rbn_b) - mean2 * a2
    lo3 = a2 * jnp.min(bmin, axis=0) + c2
    hi3 = a2 * jnp.max(bmax, axis=0) + c2
    mn3 = jnp.maximum(jnp.min(lo3), 0.0)
    mx3 = jnp.maximum(jnp.max(hi3), 0.0)
    sc3 = jnp.maximum((mx3 - mn3) / QMAX, 1e-8)
    inv3 = 1.0 / sc3

    w2q = _qdq(conv2_w)
    w2t = w2q.transpose(2, 3, 1, 0).reshape(K2, GROWTH).astype(jnp.bfloat16)
    qv3 = jnp.stack([mn3 * inv3, sc3, mn3])

    hspec = lambda fn: pl.BlockSpec((RB, C_MID), fn)
    out_t = pl.pallas_call(
        _conv2_kernel,
        grid=(NB,),
        in_specs=[
            pl.BlockSpec((RB, C_IN), lambda i: (i, 0)),
            hspec(lambda i: (jnp.maximum(i - 1, 0), 0)),
            hspec(lambda i: (i, 0)),
            hspec(lambda i: (jnp.minimum(i + 1, NB - 1), 0)),
            pl.BlockSpec((1, C_MID), lambda i: (0, 0)),
            pl.BlockSpec((1, C_MID), lambda i: (0, 0)),
            pl.BlockSpec((K2, GROWTH), lambda i: (0, 0)),
            pl.BlockSpec(memory_space=pltpu.SMEM),
        ],
        out_specs=pl.BlockSpec((RB, C_OUT), lambda i: (i, 0)),
        out_shape=jax.ShapeDtypeStruct((R_ALL, C_OUT), f32),
        scratch_shapes=[
            pltpu.VMEM((SROWS, C_MID), jnp.bfloat16),
            pltpu.VMEM((RB, K2), jnp.bfloat16),
        ],
        compiler_params=pltpu.CompilerParams(
            dimension_semantics=("arbitrary",),
            vmem_limit_bytes=56 * 1024 * 1024),
        name="dense_conv2",
    )(xt, h2, h2, h2, rvec(a2 * inv3), rvec(c2 * inv3), w2t, qv3)

    return out_t.reshape(H, W, B, C_OUT).transpose(2, 3, 0, 1)


# speedup vs baseline: 3.8272x; 1.1455x over previous
import numpy as np

import jax
import jax.numpy as jnp
from jax.experimental import pallas as pl
from jax.experimental.pallas import tpu as pltpu

B, C_IN, H, W = 64, 512, 28, 28
HW = H * W
R_ALL = H * W * B
C_MID, GROWTH = 128, 32
C_OUT = C_IN + GROWTH
EPS = 1e-5
QMAX = 255.0
NUM_CHUNKS = 16
RB = 3584
NB = R_ALL // RB
HL = W * B
MARG = HL + B
SROWS = MARG + RB + MARG
K2 = 9 * C_MID
SCALE_FIX = float((0.5 * 0.35) * (1.0 + (np.pi * np.log(4.0)) ** 0.5)
                  / ((2.0 * np.log(R_ALL / NUM_CHUNKS)) ** 0.5))


def _qparams(x):
    mn = x.min()
    mx = x.max()
    scale = jnp.maximum((mx - mn) / QMAX, 1e-8)
    return mn, scale


def _stats_kernel(x_ref, o_ref):
    xb = x_ref[...]
    o_ref[0, 0] = jnp.sum(xb, axis=0)
    o_ref[0, 1] = jnp.sum(xb * xb, axis=0)
    o_ref[0, 2] = jnp.min(xb, axis=0)
    o_ref[0, 3] = jnp.max(xb, axis=0)


def _fused_kernel(x_ref, ap_ref, tp_ref, w1_ref, rc_ref, rw_ref, rb_ref,
                  w2_ref, q_ref, o_ref, h2_scr, st_scr, pr_scr, g_scr):
    p = pl.program_id(0)
    i = pl.program_id(1)
    m1p, c1, c2 = q_ref[0], q_ref[1], q_ref[2]
    bf16 = jnp.bfloat16

    @pl.when(p == 0)
    def _phase0():
        xb = x_ref[...]
        v = jnp.maximum(xb * ap_ref[...] + tp_ref[...], 0.0) - m1p
        q1 = jnp.clip(jnp.round(v), 0.0, QMAX).astype(bf16)
        s = jnp.dot(q1, w1_ref[...],
                    preferred_element_type=jnp.float32)
        h2 = c1 * s[:, :C_MID] + c2 * s[:, C_MID:C_MID + 1] + rc_ref[...]
        h2b = h2.astype(bf16)
        base = pl.multiple_of(MARG + i * RB, B)
        h2_scr[pl.ds(base, HL)] = h2b[:HL]
        h2_scr[pl.ds(base + HL, HL)] = h2b[HL:]
        h2r = h2b.astype(jnp.float32).reshape(RB // B, B, C_MID)
        s_k = jnp.sum(h2r, axis=0)
        mn_k = jnp.min(h2r, axis=0)
        mx_k = jnp.max(h2r, axis=0)

        @pl.when(i == 0)
        def _init():
            st_scr[0] = s_k
            st_scr[1] = mn_k
            st_scr[2] = mx_k

        @pl.when(i > 0)
        def _acc():
            st_scr[0] += s_k
            st_scr[1] = jnp.minimum(st_scr[1], mn_k)
            st_scr[2] = jnp.maximum(st_scr[2], mx_k)

    @pl.when(p == 1)
    def _phase1():
        @pl.when(i == 0)
        def _params():
            bsum, bmin, bmax = st_scr[0], st_scr[1], st_scr[2]
            mean2 = jnp.sum(bsum, axis=0, keepdims=True) / float(R_ALL)
            cmn = jnp.min(bmin.reshape(NUM_CHUNKS, 4, C_MID), axis=1)
            cmx = jnp.max(bmax.reshape(NUM_CHUNKS, 4, C_MID), axis=1)
            mean_min = jnp.mean(cmn, axis=0, keepdims=True)
            mean_max = jnp.mean(cmx, axis=0, keepdims=True)
            scale2 = 1.0 / ((mean_max - mean_min) * SCALE_FIX + EPS)

            def qdq(t):
                mn = jnp.min(t, axis=1, keepdims=True)
                mx = jnp.max(t, axis=1, keepdims=True)
                sc = jnp.maximum((mx - mn) / QMAX, 1e-8)
                return jnp.round((t - mn) / sc) * sc + mn

            a2 = scale2 * qdq(rw_ref[...])
            c2v = qdq(rb_ref[...]) - mean2 * a2
            gmn = jnp.min(bmin, axis=0, keepdims=True)
            gmx = jnp.max(bmax, axis=0, keepdims=True)
            lo3 = a2 * gmn + c2v
            hi3 = a2 * gmx + c2v
            mn3 = jnp.maximum(jnp.min(lo3, axis=1, keepdims=True), 0.0)
            mx3 = jnp.maximum(jnp.max(hi3, axis=1, keepdims=True), 0.0)
            sc3 = jnp.maximum((mx3 - mn3) / QMAX, 1e-8)
            inv3 = 1.0 / sc3
            pr_scr[0:1] = a2 * inv3
            pr_scr[1:2] = c2v * inv3
            pr_scr[2:3] = jnp.broadcast_to(mn3 * inv3, (1, C_MID))
            pr_scr[3:4] = jnp.broadcast_to(sc3, (1, C_MID))
            pr_scr[4:5] = jnp.broadcast_to(mn3, (1, C_MID))

        a3 = pr_scr[0:1]
        c3 = pr_scr[1:2]
        m3p = pr_scr[2:3]
        sc3v = pr_scr[3:4]
        mn3v = pr_scr[4:5]

        def quant_rows(base, n):
            raw = h2_scr[pl.ds(base, n)].astype(jnp.float32)
            v = jnp.maximum(raw * a3 + c3, 0.0) - m3p
            q3 = jnp.clip(jnp.round(v), 0.0, QMAX)
            h2_scr[pl.ds(base, n)] = (q3 * sc3v + mn3v).astype(bf16)

        @pl.when(i == 0)
        def _qhead():
            quant_rows(MARG, MARG)

        @pl.when(i < NB - 1)
        def _qmain():
            quant_rows(pl.multiple_of(MARG + i * RB + MARG, B), RB)

        @pl.when(i == NB - 1)
        def _qtail():
            quant_rows(MARG + (NB - 1) * RB + MARG,
                       R_ALL - (NB - 1) * RB - MARG)

        for o_i in range(9):
            dy, dx = o_i // 3 - 1, o_i % 3 - 1
            base = pl.multiple_of(MARG + i * RB + dy * HL + dx * B, B)
            g_scr[:, o_i * C_MID:(o_i + 1) * C_MID] = h2_scr[pl.ds(base, RB)]

        zb = jnp.zeros((B, C_MID), bf16)
        for o_i in (0, 3, 6):
            oc = slice(o_i * C_MID, (o_i + 1) * C_MID)
            g_scr[0:B, oc] = zb
            g_scr[HL:HL + B, oc] = zb
        for o_i in (2, 5, 8):
            oc = slice(o_i * C_MID, (o_i + 1) * C_MID)
            g_scr[HL - B:HL, oc] = zb
            g_scr[RB - B:RB, oc] = zb

        @pl.when(i == 0)
        def _first():
            g_scr[0:HL, 0:3 * C_MID] = jnp.zeros((HL, 3 * C_MID), bf16)

        @pl.when(i == NB - 1)
        def _last():
            g_scr[HL:RB, 6 * C_MID:] = jnp.zeros((HL, 3 * C_MID), bf16)

        h4 = jnp.dot(g_scr[...], w2_ref[...],
                     preferred_element_type=jnp.float32)
        o_ref[:, :C_IN] = x_ref[...]
        o_ref[:, C_IN:] = h4


def kernel(x, bn1_w, bn1_b, conv1_w, rbn_w, rbn_b, conv2_w):
    xt = x.transpose(2, 3, 0, 1).reshape(R_ALL, C_IN)
    f32 = jnp.float32

    st1 = pl.pallas_call(
        _stats_kernel,
        grid=(NB,),
        in_specs=[pl.BlockSpec((RB, C_IN), lambda i: (i, 0))],
        out_specs=pl.BlockSpec((1, 4, C_IN), lambda i: (i, 0, 0)),
        out_shape=jax.ShapeDtypeStruct((NB, 4, C_IN), f32),
        compiler_params=pltpu.CompilerParams(
            dimension_semantics=("arbitrary",),
            vmem_limit_bytes=56 * 1024 * 1024),
        name="dense_stats",
    )(xt)

    n1 = float(R_ALL)
    mean1 = jnp.sum(st1[:, 0], axis=0) / n1
    var1 = jnp.sum(st1[:, 1], axis=0) / n1 - mean1 * mean1
    a1 = bn1_w * jax.lax.rsqrt(var1 + EPS)
    t1 = bn1_b - mean1 * a1
    lo = a1 * jnp.min(st1[:, 2], axis=0) + t1
    hi = a1 * jnp.max(st1[:, 3], axis=0) + t1
    mn1 = jnp.maximum(jnp.min(lo), 0.0)
    mx1 = jnp.maximum(jnp.max(hi), 0.0)
    sc1 = jnp.maximum((mx1 - mn1) / QMAX, 1e-8)
    inv1 = 1.0 / sc1

    w1 = conv1_w.reshape(C_MID, C_IN)
    mnw1, sw1 = _qparams(w1)
    qw1 = jnp.round((w1 - mnw1) / sw1)
    w1aug = jnp.concatenate(
        [qw1.T, jnp.ones((C_IN, 1), f32), jnp.zeros((C_IN, 7), f32)],
        axis=1).astype(jnp.bfloat16)
    rc = (sw1 * mn1) * jnp.sum(qw1, axis=1) + (C_IN * mnw1) * mn1
    qv1 = jnp.stack([mn1 * inv1, sw1 * sc1, mnw1 * sc1])

    def _qdq(t):
        mn, sc = _qparams(t)
        return jnp.round((t - mn) / sc) * sc + mn

    w2t = _qdq(conv2_w).transpose(2, 3, 1, 0).reshape(
        K2, GROWTH).astype(jnp.bfloat16)

    rvec = lambda v: v.reshape(1, -1)
    full = lambda n: pl.BlockSpec((1, n), lambda p, i: (0, 0))
    out_t = pl.pallas_call(
        _fused_kernel,
        grid=(2, NB),
        in_specs=[
            pl.BlockSpec((RB, C_IN), lambda p, i: (i, 0)),
            full(C_IN), full(C_IN),
            pl.BlockSpec((C_IN, C_MID + 8), lambda p, i: (0, 0)),
            full(C_MID), full(C_MID), full(C_MID),
            pl.BlockSpec((K2, GROWTH), lambda p, i: (0, 0)),
            pl.BlockSpec(memory_space=pltpu.SMEM),
        ],
        out_specs=pl.BlockSpec((RB, C_OUT), lambda p, i: (p * i, 0)),
        out_shape=jax.ShapeDtypeStruct((R_ALL, C_OUT), f32),
        scratch_shapes=[
            pltpu.VMEM((MARG + R_ALL + MARG, C_MID), jnp.bfloat16),
            pltpu.VMEM((3, B, C_MID), f32),
            pltpu.VMEM((5, C_MID), f32),
            pltpu.VMEM((RB, K2), jnp.bfloat16),
        ],
        compiler_params=pltpu.CompilerParams(
            dimension_semantics=("arbitrary", "arbitrary"),
            vmem_limit_bytes=56 * 1024 * 1024),
        name="dense_fused",
    )(xt, rvec(a1 * inv1), rvec(t1 * inv1), w1aug, rvec(rc),
      rvec(rbn_w), rvec(rbn_b),
      w2t, qv1)

    return out_t.reshape(H, W, B, C_OUT).transpose(2, 3, 0, 1)
